# baseline (speedup 1.0000x reference)
import functools

import jax
import jax.numpy as jnp
from jax import lax
from jax.experimental import pallas as pl
from jax.experimental.pallas import tpu as pltpu
from jax.experimental.pallas import tpu_sc as plsc

B = 16384
K = 32
NC = 2
NS = 16
L = 16
NW = NC * NS
BPW = B // NW
CHK = 16
NCHK = BPW // CHK
SLAB = 8


def _body(uid_hbm, iid_hbm, pi_hbm, eta_hbm, out_hbm,
          uid_v, iid_v, out_v, stash_v,
          pi_a, pi_b, eta_a, eta_b, sem_a, sem_b):
    wid = lax.axis_index("s") * NC + lax.axis_index("c")

    pltpu.sync_copy(uid_hbm.at[wid], uid_v)
    pltpu.sync_copy(iid_hbm.at[wid], iid_v)

    last_lane = lax.iota(jnp.int32, L) * L + (L - 1)

    def fire(c, pi_buf, eta_buf, sem):
        uvec = uid_v[pl.ds(c * CHK, CHK)]
        tvec = iid_v[pl.ds(c * CHK, CHK)]
        for j in range(CHK):
            ub = pl.multiple_of(uvec[j] & ~(SLAB - 1), SLAB)
            tb = pl.multiple_of(tvec[j] & ~(SLAB - 1), SLAB)
            pltpu.async_copy(
                pi_hbm.at[pl.ds(ub, SLAB)],
                pi_buf.at[pl.ds(j * SLAB, SLAB)], sem)
            pltpu.async_copy(
                eta_hbm.at[pl.ds(tb, SLAB)],
                eta_buf.at[pl.ds(j * SLAB, SLAB)], sem)

    def drain(pi_buf, eta_buf, sem):
        pltpu.make_async_copy(
            pi_hbm.at[pl.ds(0, CHK * SLAB)], pi_buf, sem).wait()
        pltpu.make_async_copy(
            eta_hbm.at[pl.ds(0, CHK * SLAB)], eta_buf, sem).wait()

    def compute(c, pi_buf, eta_buf):
        uvec = uid_v[pl.ds(c * CHK, CHK)]
        tvec = iid_v[pl.ds(c * CHK, CHK)]
        for j in range(CHK):
            r = j * SLAB + (uvec[j] & (SLAB - 1))
            s = j * SLAB + (tvec[j] & (SLAB - 1))
            v = (pi_buf[r, pl.ds(0, L)] * eta_buf[s, pl.ds(0, L)]
                 + pi_buf[r, pl.ds(L, L)] * eta_buf[s, pl.ds(L, L)])
            stash_v[pl.ds(j * L, L)] = plsc.cumsum(v)
        sums = plsc.load_gather(stash_v, [last_lane])
        out_v[pl.ds(c * CHK, CHK)] = 1.0 - jnp.exp(-sums)

    fire(0, pi_a, eta_a, sem_a)

    def pair(p, carry):
        c0 = p * 2
        fire(c0 + 1, pi_b, eta_b, sem_b)
        drain(pi_a, eta_a, sem_a)
        compute(c0, pi_a, eta_a)

        @pl.when(p < NCHK // 2 - 1)
        def _():
            fire(c0 + 2, pi_a, eta_a, sem_a)

        drain(pi_b, eta_b, sem_b)
        compute(c0 + 1, pi_b, eta_b)
        return carry

    lax.fori_loop(0, NCHK // 2, pair, 0)

    pltpu.sync_copy(out_v, out_hbm.at[pl.ds(wid * BPW, BPW)])


_pf = functools.partial(
    pl.kernel,
    mesh=plsc.VectorSubcoreMesh(core_axis_name="c", subcore_axis_name="s"),
    out_type=jax.ShapeDtypeStruct((B,), jnp.float32),
    compiler_params=pltpu.CompilerParams(needs_layout_passes=False),
    scratch_types=[
        pltpu.VMEM((BPW,), jnp.int32),
        pltpu.VMEM((BPW,), jnp.int32),
        pltpu.VMEM((BPW,), jnp.float32),
        pltpu.VMEM((CHK * L,), jnp.float32),
        pltpu.VMEM((CHK * SLAB, K), jnp.float32),
        pltpu.VMEM((CHK * SLAB, K), jnp.float32),
        pltpu.VMEM((CHK * SLAB, K), jnp.float32),
        pltpu.VMEM((CHK * SLAB, K), jnp.float32),
        pltpu.SemaphoreType.DMA,
        pltpu.SemaphoreType.DMA,
    ],
)(_body)


def kernel(user_ids, item_ids, pi, eta):
    uid = user_ids.astype(jnp.int32).reshape(NW, BPW)
    iid = item_ids.astype(jnp.int32).reshape(NW, BPW)
    return _pf(uid, iid, pi, eta)
